# baseline (speedup 1.0000x reference)
import functools

import jax
import jax.numpy as jnp
from jax.experimental import pallas as pl

B, OBS_DIM, ACT_DIM, NUM_ACTIONS, HID, KNN = 1024, 128, 32, 100000, 256, 50
STD_NOISE, CLIP_NOISE, ACT_LO, ACT_HI = 0.1, 0.5, -1.0, 1.0

N_PAD = 102400
DIST_BLK = 2048
NBLK = N_PAD // DIST_BLK
GRP = 16
NGRP = N_PAD // GRP
PAD_VAL = 1000.0


def _actor_body(obs, noise, w1, b1, w2, b2, w3, b3, proto_out):
    h = jax.nn.relu(jnp.dot(obs[...], w1[...]) + b1[...])
    h = jax.nn.relu(jnp.dot(h, w2[...]) + b2[...])
    p = jnp.tanh(jnp.dot(h, w3[...]) + b3[...])
    smooth = jnp.clip(noise[...] * STD_NOISE, -CLIP_NOISE, CLIP_NOISE)
    proto_out[...] = jnp.clip(p + smooth, ACT_LO, ACT_HI)


def _dist_body(proto, psq, table, asq, d_out, gmin_out):
    s = jnp.dot(proto[...], table[...].T)
    d = psq[...] - 2.0 * s + asq[...]
    d_out[...] = d
    gmin_out[...] = jnp.min(d.reshape(B, GRP, 128), axis=1)


def _select_body(g_ref, gid_out):
    blk = g_ref.shape[0]
    g = g_ref[...]
    iota = jax.lax.broadcasted_iota(jnp.int32, (blk, NGRP), 1)
    out_iota = jax.lax.broadcasted_iota(jnp.int32, (blk, 128), 1)

    def step(i, carry):
        g, gids = carry
        m = jnp.min(g, axis=1, keepdims=True)
        loc = jnp.min(jnp.where(g == m, iota, NGRP), axis=1, keepdims=True)
        g = jnp.where(iota == loc, jnp.float32(jnp.inf), g)
        gids = jnp.where(out_iota == i, loc, gids)
        return g, gids

    _, gids = jax.lax.fori_loop(
        0, KNN, step, (g, jnp.zeros((blk, 128), jnp.int32)))
    gid_out[...] = gids


def _critic_body(obs, acts, w1, b1, w2, b2, w3, b3, best_out):
    blk = obs.shape[0]
    obs_t = jnp.broadcast_to(obs[...][:, None, :], (blk, KNN, OBS_DIM))
    ca = jnp.concatenate([obs_t, acts[...]], axis=-1)
    ca = ca.reshape(blk * KNN, OBS_DIM + ACT_DIM)
    q = jax.nn.relu(jnp.dot(ca, w1[...]) + b1[...])
    q = jax.nn.relu(jnp.dot(q, w2[...]) + b2[...])
    q = (jnp.dot(q, w3[...]) + b3[...]).reshape(blk, KNN)
    iota = jax.lax.broadcasted_iota(jnp.int32, (blk, KNN), 1)
    m = jnp.max(q, axis=1, keepdims=True)
    sel = jnp.where(q >= m, iota, KNN)
    bi = jnp.min(sel, axis=1, keepdims=True)
    onehot = (iota == bi).astype(jnp.float32)
    best_out[...] = jnp.sum(acts[...] * onehot[:, :, None], axis=1)


def kernel(obs, noise, actions_table, W1a, b1a, W2a, b2a, W3a, b3a,
           W1c, b1c, W2c, b2c, W3c, b3c):
    f32 = jnp.float32

    proto = pl.pallas_call(
        _actor_body,
        out_shape=jax.ShapeDtypeStruct((B, ACT_DIM), f32),
    )(obs, noise, W1a, b1a.reshape(1, HID), W2a, b2a.reshape(1, HID),
      W3a, b3a.reshape(1, ACT_DIM))

    table_pad = jnp.concatenate(
        [actions_table,
         jnp.full((N_PAD - NUM_ACTIONS, ACT_DIM), PAD_VAL, f32)], axis=0)
    psq = jnp.sum(proto * proto, axis=1, keepdims=True)
    asq = jnp.sum(table_pad * table_pad, axis=1)[None, :]

    d, gmin = pl.pallas_call(
        _dist_body,
        grid=(NBLK,),
        in_specs=[
            pl.BlockSpec((B, ACT_DIM), lambda i: (0, 0)),
            pl.BlockSpec((B, 1), lambda i: (0, 0)),
            pl.BlockSpec((DIST_BLK, ACT_DIM), lambda i: (i, 0)),
            pl.BlockSpec((1, DIST_BLK), lambda i: (0, i)),
        ],
        out_specs=[
            pl.BlockSpec((B, DIST_BLK), lambda i: (0, i)),
            pl.BlockSpec((B, 128), lambda i: (0, i)),
        ],
        out_shape=[
            jax.ShapeDtypeStruct((B, N_PAD), f32),
            jax.ShapeDtypeStruct((B, NGRP), f32),
        ],
    )(proto, psq, table_pad, asq)

    sblk = 128
    gidx_full = pl.pallas_call(
        _select_body,
        grid=(B // sblk,),
        in_specs=[pl.BlockSpec((sblk, NGRP), lambda i: (i, 0))],
        out_specs=pl.BlockSpec((sblk, 128), lambda i: (i, 0)),
        out_shape=jax.ShapeDtypeStruct((B, 128), jnp.int32),
    )(gmin)
    gidx = gidx_full[:, :KNN]

    dcols = d.reshape(B, NBLK, GRP, 128).transpose(0, 1, 3, 2)
    dcols = dcols.reshape(B, NGRP, GRP)
    cands = jnp.take_along_axis(dcols, gidx[:, :, None], axis=1)
    c_idx = gidx // 128
    l_idx = gidx % 128
    base = c_idx * DIST_BLK + l_idx
    nn = base[:, :, None] + 128 * jnp.arange(GRP, dtype=jnp.int32)[None, None, :]
    cv = cands.reshape(B, KNN * GRP)
    ni = nn.reshape(B, KNN * GRP)
    _, sel = jax.lax.top_k(-cv, KNN)
    idx = jnp.take_along_axis(ni, sel, axis=1)
    raw_actions = jnp.take(actions_table, idx, axis=0)

    bblk = 128
    best = pl.pallas_call(
        _critic_body,
        grid=(B // bblk,),
        in_specs=[
            pl.BlockSpec((bblk, OBS_DIM), lambda i: (i, 0)),
            pl.BlockSpec((bblk, KNN, ACT_DIM), lambda i: (i, 0, 0)),
            pl.BlockSpec((OBS_DIM + ACT_DIM, HID), lambda i: (0, 0)),
            pl.BlockSpec((1, HID), lambda i: (0, 0)),
            pl.BlockSpec((HID, HID), lambda i: (0, 0)),
            pl.BlockSpec((1, HID), lambda i: (0, 0)),
            pl.BlockSpec((HID, 1), lambda i: (0, 0)),
            pl.BlockSpec((1, 1), lambda i: (0, 0)),
        ],
        out_specs=pl.BlockSpec((bblk, ACT_DIM), lambda i: (i, 0)),
        out_shape=jax.ShapeDtypeStruct((B, ACT_DIM), f32),
    )(obs, raw_actions, W1c, b1c.reshape(1, HID), W2c, b2c.reshape(1, HID),
      W3c, b3c.reshape(1, 1))
    return best


# speedup vs baseline: 1.4239x; 1.4239x over previous
import functools

import jax
import jax.numpy as jnp
from jax.experimental import pallas as pl

B, OBS_DIM, ACT_DIM, NUM_ACTIONS, HID, KNN = 1024, 128, 32, 100000, 256, 50
STD_NOISE, CLIP_NOISE, ACT_LO, ACT_HI = 0.1, 0.5, -1.0, 1.0

N_PAD = 102400
DIST_BLK = 2048
NBLK = N_PAD // DIST_BLK
GRP = 16
NGRP = N_PAD // GRP
PAD_VAL = 1000.0


def _actor_body(obs, noise, w1, b1, w2, b2, w3, b3, proto_out):
    h = jax.nn.relu(jnp.dot(obs[...], w1[...]) + b1[...])
    h = jax.nn.relu(jnp.dot(h, w2[...]) + b2[...])
    p = jnp.tanh(jnp.dot(h, w3[...]) + b3[...])
    smooth = jnp.clip(noise[...] * STD_NOISE, -CLIP_NOISE, CLIP_NOISE)
    proto_out[...] = jnp.clip(p + smooth, ACT_LO, ACT_HI)


def _dist_body(proto, psq, table, asq, dcols_out, gmin_out):
    s = jnp.dot(proto[...], table[...].T)
    d = psq[...] - 2.0 * s + asq[...]
    d3 = d.reshape(B, GRP, 128)
    dcols_out[...] = d3
    gmin_out[...] = jnp.min(d3, axis=1)


def _select_body(g_ref, gid_out):
    blk = g_ref.shape[0]
    g = g_ref[...]
    iota = jax.lax.broadcasted_iota(jnp.int32, (blk, NGRP), 1)
    out_iota = jax.lax.broadcasted_iota(jnp.int32, (blk, 128), 1)

    def step(i, carry):
        g, gids = carry
        m = jnp.min(g, axis=1, keepdims=True)
        loc = jnp.min(jnp.where(g == m, iota, NGRP), axis=1, keepdims=True)
        g = jnp.where(iota == loc, jnp.float32(jnp.inf), g)
        gids = jnp.where(out_iota == i, loc, gids)
        return g, gids

    _, gids = jax.lax.fori_loop(
        0, KNN, step, (g, jnp.zeros((blk, 128), jnp.int32)))
    gid_out[...] = gids


def _critic_body(obs, acts, w1, b1, w2, b2, w3, b3, best_out):
    blk = obs.shape[0]
    obs_t = jnp.broadcast_to(obs[...][:, None, :], (blk, KNN, OBS_DIM))
    ca = jnp.concatenate([obs_t, acts[...]], axis=-1)
    ca = ca.reshape(blk * KNN, OBS_DIM + ACT_DIM)
    q = jax.nn.relu(jnp.dot(ca, w1[...]) + b1[...])
    q = jax.nn.relu(jnp.dot(q, w2[...]) + b2[...])
    q = (jnp.dot(q, w3[...]) + b3[...]).reshape(blk, KNN)
    iota = jax.lax.broadcasted_iota(jnp.int32, (blk, KNN), 1)
    m = jnp.max(q, axis=1, keepdims=True)
    sel = jnp.where(q >= m, iota, KNN)
    bi = jnp.min(sel, axis=1, keepdims=True)
    onehot = (iota == bi).astype(jnp.float32)
    best_out[...] = jnp.sum(acts[...] * onehot[:, :, None], axis=1)


def kernel(obs, noise, actions_table, W1a, b1a, W2a, b2a, W3a, b3a,
           W1c, b1c, W2c, b2c, W3c, b3c):
    f32 = jnp.float32

    proto = pl.pallas_call(
        _actor_body,
        out_shape=jax.ShapeDtypeStruct((B, ACT_DIM), f32),
    )(obs, noise, W1a, b1a.reshape(1, HID), W2a, b2a.reshape(1, HID),
      W3a, b3a.reshape(1, ACT_DIM))

    table_pad = jnp.concatenate(
        [actions_table,
         jnp.full((N_PAD - NUM_ACTIONS, ACT_DIM), PAD_VAL, f32)], axis=0)
    psq = jnp.sum(proto * proto, axis=1, keepdims=True)
    asq = jnp.sum(table_pad * table_pad, axis=1)[None, :]

    dcols, gmin = pl.pallas_call(
        _dist_body,
        grid=(NBLK,),
        in_specs=[
            pl.BlockSpec((B, ACT_DIM), lambda i: (0, 0)),
            pl.BlockSpec((B, 1), lambda i: (0, 0)),
            pl.BlockSpec((DIST_BLK, ACT_DIM), lambda i: (i, 0)),
            pl.BlockSpec((1, DIST_BLK), lambda i: (0, i)),
        ],
        out_specs=[
            pl.BlockSpec((B, GRP, 128), lambda i: (0, 0, i)),
            pl.BlockSpec((B, 128), lambda i: (0, i)),
        ],
        out_shape=[
            jax.ShapeDtypeStruct((B, GRP, NGRP), f32),
            jax.ShapeDtypeStruct((B, NGRP), f32),
        ],
    )(proto, psq, table_pad, asq)

    sblk = 128
    gidx_full = pl.pallas_call(
        _select_body,
        grid=(B // sblk,),
        in_specs=[pl.BlockSpec((sblk, NGRP), lambda i: (i, 0))],
        out_specs=pl.BlockSpec((sblk, 128), lambda i: (i, 0)),
        out_shape=jax.ShapeDtypeStruct((B, 128), jnp.int32),
    )(gmin)
    gidx = gidx_full[:, :KNN]

    cands = jnp.take_along_axis(dcols, gidx[:, None, :], axis=2)
    c_idx = gidx // 128
    l_idx = gidx % 128
    base = c_idx * DIST_BLK + l_idx
    nn = base[:, None, :] + 128 * jnp.arange(GRP, dtype=jnp.int32)[None, :, None]
    cv = cands.reshape(B, KNN * GRP)
    ni = nn.reshape(B, KNN * GRP)
    _, sel = jax.lax.top_k(-cv, KNN)
    idx = jnp.take_along_axis(ni, sel, axis=1)
    raw_actions = jnp.take(actions_table, idx, axis=0)

    bblk = 128
    best = pl.pallas_call(
        _critic_body,
        grid=(B // bblk,),
        in_specs=[
            pl.BlockSpec((bblk, OBS_DIM), lambda i: (i, 0)),
            pl.BlockSpec((bblk, KNN, ACT_DIM), lambda i: (i, 0, 0)),
            pl.BlockSpec((OBS_DIM + ACT_DIM, HID), lambda i: (0, 0)),
            pl.BlockSpec((1, HID), lambda i: (0, 0)),
            pl.BlockSpec((HID, HID), lambda i: (0, 0)),
            pl.BlockSpec((1, HID), lambda i: (0, 0)),
            pl.BlockSpec((HID, 1), lambda i: (0, 0)),
            pl.BlockSpec((1, 1), lambda i: (0, 0)),
        ],
        out_specs=pl.BlockSpec((bblk, ACT_DIM), lambda i: (i, 0)),
        out_shape=jax.ShapeDtypeStruct((B, ACT_DIM), f32),
    )(obs, raw_actions, W1c, b1c.reshape(1, HID), W2c, b2c.reshape(1, HID),
      W3c, b3c.reshape(1, 1))
    return best


# speedup vs baseline: 1.4247x; 1.0006x over previous
import functools

import jax
import jax.numpy as jnp
from jax.experimental import pallas as pl
from jax.experimental.pallas import tpu as pltpu

B, OBS_DIM, ACT_DIM, NUM_ACTIONS, HID, KNN = 1024, 128, 32, 100000, 256, 50
STD_NOISE, CLIP_NOISE, ACT_LO, ACT_HI = 0.1, 0.5, -1.0, 1.0

N_PAD = 102400
DIST_BLK = 2048
NBLK = N_PAD // DIST_BLK
GRP = 16
NGRP = N_PAD // GRP
PAD_VAL = 1000.0


def _actor_body(obs, noise, w1, b1, w2, b2, w3, b3, proto_out):
    h = jax.nn.relu(jnp.dot(obs[...], w1[...]) + b1[...])
    h = jax.nn.relu(jnp.dot(h, w2[...]) + b2[...])
    p = jnp.tanh(jnp.dot(h, w3[...]) + b3[...])
    smooth = jnp.clip(noise[...] * STD_NOISE, -CLIP_NOISE, CLIP_NOISE)
    proto_out[...] = jnp.clip(p + smooth, ACT_LO, ACT_HI)


def _dist_body(proto, psq, table, asq, dcols_out, gmin_out):
    s = jnp.dot(proto[...], table[...].T)
    d = psq[...] - 2.0 * s + asq[...]
    d3 = d.reshape(B, GRP, 128)
    dcols_out[...] = d3
    gmin_out[...] = jnp.min(d3, axis=1)


def _select_body(g_ref, gid_out):
    blk = g_ref.shape[0]
    g = g_ref[...]
    iota = jax.lax.broadcasted_iota(jnp.int32, (blk, NGRP), 1)
    out_iota = jax.lax.broadcasted_iota(jnp.int32, (blk, 128), 1)

    def step(i, carry):
        g, gids = carry
        m = jnp.min(g, axis=1, keepdims=True)
        loc = jnp.min(jnp.where(g == m, iota, NGRP), axis=1, keepdims=True)
        g = jnp.where(iota == loc, jnp.float32(jnp.inf), g)
        gids = jnp.where(out_iota == i, loc, gids)
        return g, gids

    _, gids = jax.lax.fori_loop(
        0, KNN, step, (g, jnp.zeros((blk, 128), jnp.int32)))
    gid_out[...] = gids


def _critic_body(obs, acts, w1, b1, w2, b2, w3, b3, best_out):
    blk = obs.shape[0]
    obs_t = jnp.broadcast_to(obs[...][:, None, :], (blk, KNN, OBS_DIM))
    ca = jnp.concatenate([obs_t, acts[...]], axis=-1)
    ca = ca.reshape(blk * KNN, OBS_DIM + ACT_DIM)
    q = jax.nn.relu(jnp.dot(ca, w1[...]) + b1[...])
    q = jax.nn.relu(jnp.dot(q, w2[...]) + b2[...])
    q = (jnp.dot(q, w3[...]) + b3[...]).reshape(blk, KNN)
    iota = jax.lax.broadcasted_iota(jnp.int32, (blk, KNN), 1)
    m = jnp.max(q, axis=1, keepdims=True)
    sel = jnp.where(q >= m, iota, KNN)
    bi = jnp.min(sel, axis=1, keepdims=True)
    onehot = (iota == bi).astype(jnp.float32)
    best_out[...] = jnp.sum(acts[...] * onehot[:, :, None], axis=1)


def kernel(obs, noise, actions_table, W1a, b1a, W2a, b2a, W3a, b3a,
           W1c, b1c, W2c, b2c, W3c, b3c):
    f32 = jnp.float32

    proto = pl.pallas_call(
        _actor_body,
        out_shape=jax.ShapeDtypeStruct((B, ACT_DIM), f32),
    )(obs, noise, W1a, b1a.reshape(1, HID), W2a, b2a.reshape(1, HID),
      W3a, b3a.reshape(1, ACT_DIM))

    table_pad = jnp.concatenate(
        [actions_table,
         jnp.full((N_PAD - NUM_ACTIONS, ACT_DIM), PAD_VAL, f32)], axis=0)
    psq = jnp.sum(proto * proto, axis=1, keepdims=True)
    asq = jnp.sum(table_pad * table_pad, axis=1)[None, :]

    dcols, gmin = pl.pallas_call(
        _dist_body,
        grid=(NBLK,),
        in_specs=[
            pl.BlockSpec((B, ACT_DIM), lambda i: (0, 0)),
            pl.BlockSpec((B, 1), lambda i: (0, 0)),
            pl.BlockSpec((DIST_BLK, ACT_DIM), lambda i: (i, 0)),
            pl.BlockSpec((1, DIST_BLK), lambda i: (0, i)),
        ],
        out_specs=[
            pl.BlockSpec((B, GRP, 128), lambda i: (0, 0, i)),
            pl.BlockSpec((B, 128), lambda i: (0, i)),
        ],
        out_shape=[
            jax.ShapeDtypeStruct((B, GRP, NGRP), f32),
            jax.ShapeDtypeStruct((B, NGRP), f32),
        ],
        compiler_params=pltpu.CompilerParams(
            dimension_semantics=("parallel",)),
    )(proto, psq, table_pad, asq)

    sblk = 128
    gidx_full = pl.pallas_call(
        _select_body,
        grid=(B // sblk,),
        in_specs=[pl.BlockSpec((sblk, NGRP), lambda i: (i, 0))],
        out_specs=pl.BlockSpec((sblk, 128), lambda i: (i, 0)),
        out_shape=jax.ShapeDtypeStruct((B, 128), jnp.int32),
        compiler_params=pltpu.CompilerParams(
            dimension_semantics=("parallel",)),
    )(gmin)
    gidx = gidx_full[:, :KNN]

    cands = jnp.take_along_axis(dcols, gidx[:, None, :], axis=2)
    c_idx = gidx // 128
    l_idx = gidx % 128
    base = c_idx * DIST_BLK + l_idx
    nn = base[:, None, :] + 128 * jnp.arange(GRP, dtype=jnp.int32)[None, :, None]
    cv = cands.reshape(B, KNN * GRP)
    ni = nn.reshape(B, KNN * GRP)
    _, sel = jax.lax.top_k(-cv, KNN)
    idx = jnp.take_along_axis(ni, sel, axis=1)
    raw_actions = jnp.take(actions_table, idx, axis=0)

    bblk = 128
    best = pl.pallas_call(
        _critic_body,
        grid=(B // bblk,),
        in_specs=[
            pl.BlockSpec((bblk, OBS_DIM), lambda i: (i, 0)),
            pl.BlockSpec((bblk, KNN, ACT_DIM), lambda i: (i, 0, 0)),
            pl.BlockSpec((OBS_DIM + ACT_DIM, HID), lambda i: (0, 0)),
            pl.BlockSpec((1, HID), lambda i: (0, 0)),
            pl.BlockSpec((HID, HID), lambda i: (0, 0)),
            pl.BlockSpec((1, HID), lambda i: (0, 0)),
            pl.BlockSpec((HID, 1), lambda i: (0, 0)),
            pl.BlockSpec((1, 1), lambda i: (0, 0)),
        ],
        out_specs=pl.BlockSpec((bblk, ACT_DIM), lambda i: (i, 0)),
        out_shape=jax.ShapeDtypeStruct((B, ACT_DIM), f32),
        compiler_params=pltpu.CompilerParams(
            dimension_semantics=("parallel",)),
    )(obs, raw_actions, W1c, b1c.reshape(1, HID), W2c, b2c.reshape(1, HID),
      W3c, b3c.reshape(1, 1))
    return best


# speedup vs baseline: 1.7614x; 1.2363x over previous
import functools

import jax
import jax.numpy as jnp
from jax.experimental import pallas as pl
from jax.experimental.pallas import tpu as pltpu

B, OBS_DIM, ACT_DIM, NUM_ACTIONS, HID, KNN = 1024, 128, 32, 100000, 256, 50
STD_NOISE, CLIP_NOISE, ACT_LO, ACT_HI = 0.1, 0.5, -1.0, 1.0

N_PAD = 102400
DIST_BLK = 2048
NBLK = N_PAD // DIST_BLK
GRP = 16
NGRP = N_PAD // GRP
NPAIR = NGRP // 2
NCAND = 2 * KNN * GRP
PAD_VAL = 1000.0


def _actor_body(obs, noise, w1, b1, w2, b2, w3, b3, proto_out):
    h = jax.nn.relu(jnp.dot(obs[...], w1[...]) + b1[...])
    h = jax.nn.relu(jnp.dot(h, w2[...]) + b2[...])
    p = jnp.tanh(jnp.dot(h, w3[...]) + b3[...])
    smooth = jnp.clip(noise[...] * STD_NOISE, -CLIP_NOISE, CLIP_NOISE)
    proto_out[...] = jnp.clip(p + smooth, ACT_LO, ACT_HI)


def _dist_body(proto, psq, table, asq, dcols_out, gmin_out):
    s = jnp.dot(proto[...], table[...].T)
    d = psq[...] - 2.0 * s + asq[...]
    d3 = d.reshape(B, GRP, 128)
    dcols_out[...] = d3
    gmin_out[...] = jnp.min(d3, axis=1)


def _select_body(g_ref, gid_out):
    blk, width = g_ref.shape
    g = g_ref[...]
    iota = jax.lax.broadcasted_iota(jnp.int32, (blk, width), 1)
    out_iota = jax.lax.broadcasted_iota(jnp.int32, (blk, 128), 1)

    def step(i, carry):
        g, gids = carry
        m = jnp.min(g, axis=1, keepdims=True)
        loc = jnp.min(jnp.where(g == m, iota, width), axis=1, keepdims=True)
        g = jnp.where(iota == loc, jnp.float32(jnp.inf), g)
        gids = jnp.where(out_iota == i, loc, gids)
        return g, gids

    _, gids = jax.lax.fori_loop(
        0, KNN, step, (g, jnp.zeros((blk, 128), jnp.int32)))
    gid_out[...] = gids


def _critic_body(obs, acts, w1, b1, w2, b2, w3, b3, best_out):
    blk = obs.shape[0]
    obs_t = jnp.broadcast_to(obs[...][:, None, :], (blk, KNN, OBS_DIM))
    ca = jnp.concatenate([obs_t, acts[...]], axis=-1)
    ca = ca.reshape(blk * KNN, OBS_DIM + ACT_DIM)
    q = jax.nn.relu(jnp.dot(ca, w1[...]) + b1[...])
    q = jax.nn.relu(jnp.dot(q, w2[...]) + b2[...])
    q = (jnp.dot(q, w3[...]) + b3[...]).reshape(blk, KNN)
    iota = jax.lax.broadcasted_iota(jnp.int32, (blk, KNN), 1)
    m = jnp.max(q, axis=1, keepdims=True)
    sel = jnp.where(q >= m, iota, KNN)
    bi = jnp.min(sel, axis=1, keepdims=True)
    onehot = (iota == bi).astype(jnp.float32)
    best_out[...] = jnp.sum(acts[...] * onehot[:, :, None], axis=1)


def kernel(obs, noise, actions_table, W1a, b1a, W2a, b2a, W3a, b3a,
           W1c, b1c, W2c, b2c, W3c, b3c):
    f32 = jnp.float32

    proto = pl.pallas_call(
        _actor_body,
        out_shape=jax.ShapeDtypeStruct((B, ACT_DIM), f32),
    )(obs, noise, W1a, b1a.reshape(1, HID), W2a, b2a.reshape(1, HID),
      W3a, b3a.reshape(1, ACT_DIM))

    table_pad = jnp.concatenate(
        [actions_table,
         jnp.full((N_PAD - NUM_ACTIONS, ACT_DIM), PAD_VAL, f32)], axis=0)
    psq = jnp.sum(proto * proto, axis=1, keepdims=True)
    asq = jnp.sum(table_pad * table_pad, axis=1)[None, :]

    dcols, gmin = pl.pallas_call(
        _dist_body,
        grid=(NBLK,),
        in_specs=[
            pl.BlockSpec((B, ACT_DIM), lambda i: (0, 0)),
            pl.BlockSpec((B, 1), lambda i: (0, 0)),
            pl.BlockSpec((DIST_BLK, ACT_DIM), lambda i: (i, 0)),
            pl.BlockSpec((1, DIST_BLK), lambda i: (0, i)),
        ],
        out_specs=[
            pl.BlockSpec((B, GRP, 128), lambda i: (0, 0, i)),
            pl.BlockSpec((B, 128), lambda i: (0, i)),
        ],
        out_shape=[
            jax.ShapeDtypeStruct((B, GRP, NGRP), f32),
            jax.ShapeDtypeStruct((B, NGRP), f32),
        ],
        compiler_params=pltpu.CompilerParams(
            dimension_semantics=("parallel",)),
    )(proto, psq, table_pad, asq)

    pmin = jnp.minimum(gmin[:, :NPAIR], gmin[:, NPAIR:])
    sblk = 128
    pid_full = pl.pallas_call(
        _select_body,
        grid=(B // sblk,),
        in_specs=[pl.BlockSpec((sblk, NPAIR), lambda i: (i, 0))],
        out_specs=pl.BlockSpec((sblk, 128), lambda i: (i, 0)),
        out_shape=jax.ShapeDtypeStruct((B, 128), jnp.int32),
        compiler_params=pltpu.CompilerParams(
            dimension_semantics=("parallel",)),
    )(pmin)
    pid = pid_full[:, :KNN]

    gidx2 = jnp.concatenate([pid, pid + NPAIR], axis=1)
    cands = jnp.take_along_axis(dcols, gidx2[:, None, :], axis=2)
    c_idx2 = gidx2 // 128
    l_idx2 = gidx2 % 128
    base2 = c_idx2 * DIST_BLK + l_idx2
    nn = base2[:, None, :] + 128 * jnp.arange(GRP, dtype=jnp.int32)[None, :, None]
    cv = cands.reshape(B, NCAND)
    ni = nn.reshape(B, NCAND)

    pos_full = pl.pallas_call(
        _select_body,
        grid=(B // sblk,),
        in_specs=[pl.BlockSpec((sblk, NCAND), lambda i: (i, 0))],
        out_specs=pl.BlockSpec((sblk, 128), lambda i: (i, 0)),
        out_shape=jax.ShapeDtypeStruct((B, 128), jnp.int32),
        compiler_params=pltpu.CompilerParams(
            dimension_semantics=("parallel",)),
    )(cv)
    idx = jnp.take_along_axis(ni, pos_full[:, :KNN], axis=1)
    raw_actions = jnp.take(actions_table, idx, axis=0)

    bblk = 128
    best = pl.pallas_call(
        _critic_body,
        grid=(B // bblk,),
        in_specs=[
            pl.BlockSpec((bblk, OBS_DIM), lambda i: (i, 0)),
            pl.BlockSpec((bblk, KNN, ACT_DIM), lambda i: (i, 0, 0)),
            pl.BlockSpec((OBS_DIM + ACT_DIM, HID), lambda i: (0, 0)),
            pl.BlockSpec((1, HID), lambda i: (0, 0)),
            pl.BlockSpec((HID, HID), lambda i: (0, 0)),
            pl.BlockSpec((1, HID), lambda i: (0, 0)),
            pl.BlockSpec((HID, 1), lambda i: (0, 0)),
            pl.BlockSpec((1, 1), lambda i: (0, 0)),
        ],
        out_specs=pl.BlockSpec((bblk, ACT_DIM), lambda i: (i, 0)),
        out_shape=jax.ShapeDtypeStruct((B, ACT_DIM), f32),
        compiler_params=pltpu.CompilerParams(
            dimension_semantics=("parallel",)),
    )(obs, raw_actions, W1c, b1c.reshape(1, HID), W2c, b2c.reshape(1, HID),
      W3c, b3c.reshape(1, 1))
    return best


# speedup vs baseline: 2.0100x; 1.1411x over previous
import functools

import jax
import jax.numpy as jnp
from jax.experimental import pallas as pl
from jax.experimental.pallas import tpu as pltpu

B, OBS_DIM, ACT_DIM, NUM_ACTIONS, HID, KNN = 1024, 128, 32, 100000, 256, 50
STD_NOISE, CLIP_NOISE, ACT_LO, ACT_HI = 0.1, 0.5, -1.0, 1.0

N_PAD = 102400
DIST_BLK = 2048
NBLK = N_PAD // DIST_BLK
GRP = 16
NGRP = N_PAD // GRP
NPAIR = NGRP // 2
NCAND = 2 * KNN * GRP
PAD_VAL = 1000.0


def _actor_body(obs, noise, w1, b1, w2, b2, w3, b3, proto_out):
    h = jax.nn.relu(jnp.dot(obs[...], w1[...]) + b1[...])
    h = jax.nn.relu(jnp.dot(h, w2[...]) + b2[...])
    p = jnp.tanh(jnp.dot(h, w3[...]) + b3[...])
    smooth = jnp.clip(noise[...] * STD_NOISE, -CLIP_NOISE, CLIP_NOISE)
    proto_out[...] = jnp.clip(p + smooth, ACT_LO, ACT_HI)


def _dist_body(proto, psq, table, asq, dcols_out, gmin_out):
    s = jnp.dot(proto[...], table[...].T)
    d = psq[...] - 2.0 * s + asq[...]
    d3 = d.reshape(B, GRP, 128)
    dcols_out[...] = d3
    gmin_out[...] = jnp.min(d3, axis=1)


def _select_body(g_ref, gid_out):
    blk, width = g_ref.shape
    g = g_ref[...]
    iota = jax.lax.broadcasted_iota(jnp.int32, (blk, width), 1)
    out_iota = jax.lax.broadcasted_iota(jnp.int32, (blk, 128), 1)

    def step(i, carry):
        g, gids = carry
        m = jnp.min(g, axis=1, keepdims=True)
        loc = jnp.min(jnp.where(g == m, iota, width), axis=1, keepdims=True)
        g = jnp.where(iota == loc, jnp.float32(jnp.inf), g)
        gids = jnp.where(out_iota == i, loc, gids)
        return g, gids

    _, gids = jax.lax.fori_loop(
        0, KNN, step, (g, jnp.zeros((blk, 128), jnp.int32)), unroll=2)
    gid_out[...] = gids


def _critic_body(obs, acts, w1, b1, w2, b2, w3, b3, best_out):
    blk = obs.shape[0]
    obs_t = jnp.broadcast_to(obs[...][:, None, :], (blk, KNN, OBS_DIM))
    ca = jnp.concatenate([obs_t, acts[...]], axis=-1)
    ca = ca.reshape(blk * KNN, OBS_DIM + ACT_DIM)
    q = jax.nn.relu(jnp.dot(ca, w1[...]) + b1[...])
    q = jax.nn.relu(jnp.dot(q, w2[...]) + b2[...])
    q = (jnp.dot(q, w3[...]) + b3[...]).reshape(blk, KNN)
    iota = jax.lax.broadcasted_iota(jnp.int32, (blk, KNN), 1)
    m = jnp.max(q, axis=1, keepdims=True)
    sel = jnp.where(q >= m, iota, KNN)
    bi = jnp.min(sel, axis=1, keepdims=True)
    onehot = (iota == bi).astype(jnp.float32)
    best_out[...] = jnp.sum(acts[...] * onehot[:, :, None], axis=1)


def kernel(obs, noise, actions_table, W1a, b1a, W2a, b2a, W3a, b3a,
           W1c, b1c, W2c, b2c, W3c, b3c):
    f32 = jnp.float32

    proto = pl.pallas_call(
        _actor_body,
        out_shape=jax.ShapeDtypeStruct((B, ACT_DIM), f32),
    )(obs, noise, W1a, b1a.reshape(1, HID), W2a, b2a.reshape(1, HID),
      W3a, b3a.reshape(1, ACT_DIM))

    table_pad = jnp.concatenate(
        [actions_table,
         jnp.full((N_PAD - NUM_ACTIONS, ACT_DIM), PAD_VAL, f32)], axis=0)
    psq = jnp.sum(proto * proto, axis=1, keepdims=True)
    asq = jnp.sum(table_pad * table_pad, axis=1)[None, :]

    dcols, gmin = pl.pallas_call(
        _dist_body,
        grid=(NBLK,),
        in_specs=[
            pl.BlockSpec((B, ACT_DIM), lambda i: (0, 0)),
            pl.BlockSpec((B, 1), lambda i: (0, 0)),
            pl.BlockSpec((DIST_BLK, ACT_DIM), lambda i: (i, 0)),
            pl.BlockSpec((1, DIST_BLK), lambda i: (0, i)),
        ],
        out_specs=[
            pl.BlockSpec((B, GRP, 128), lambda i: (0, 0, i)),
            pl.BlockSpec((B, 128), lambda i: (0, i)),
        ],
        out_shape=[
            jax.ShapeDtypeStruct((B, GRP, NGRP), f32),
            jax.ShapeDtypeStruct((B, NGRP), f32),
        ],
        compiler_params=pltpu.CompilerParams(
            dimension_semantics=("parallel",)),
    )(proto, psq, table_pad, asq)

    pmin = jnp.minimum(gmin[:, :NPAIR], gmin[:, NPAIR:])
    sblk = 256
    pid_full = pl.pallas_call(
        _select_body,
        grid=(B // sblk,),
        in_specs=[pl.BlockSpec((sblk, NPAIR), lambda i: (i, 0))],
        out_specs=pl.BlockSpec((sblk, 128), lambda i: (i, 0)),
        out_shape=jax.ShapeDtypeStruct((B, 128), jnp.int32),
        compiler_params=pltpu.CompilerParams(
            dimension_semantics=("parallel",)),
    )(pmin)
    pid = pid_full[:, :KNN]

    gidx2 = jnp.concatenate([pid, pid + NPAIR], axis=1)
    cands = jnp.take_along_axis(dcols, gidx2[:, None, :], axis=2)
    c_idx2 = gidx2 // 128
    l_idx2 = gidx2 % 128
    base2 = c_idx2 * DIST_BLK + l_idx2
    nn = base2[:, None, :] + 128 * jnp.arange(GRP, dtype=jnp.int32)[None, :, None]
    cv = cands.reshape(B, NCAND)
    ni = nn.reshape(B, NCAND)

    pos_full = pl.pallas_call(
        _select_body,
        grid=(B // sblk,),
        in_specs=[pl.BlockSpec((sblk, NCAND), lambda i: (i, 0))],
        out_specs=pl.BlockSpec((sblk, 128), lambda i: (i, 0)),
        out_shape=jax.ShapeDtypeStruct((B, 128), jnp.int32),
        compiler_params=pltpu.CompilerParams(
            dimension_semantics=("parallel",)),
    )(cv)
    idx = jnp.take_along_axis(ni, pos_full[:, :KNN], axis=1)
    raw_actions = jnp.take(actions_table, idx, axis=0)

    bblk = 128
    best = pl.pallas_call(
        _critic_body,
        grid=(B // bblk,),
        in_specs=[
            pl.BlockSpec((bblk, OBS_DIM), lambda i: (i, 0)),
            pl.BlockSpec((bblk, KNN, ACT_DIM), lambda i: (i, 0, 0)),
            pl.BlockSpec((OBS_DIM + ACT_DIM, HID), lambda i: (0, 0)),
            pl.BlockSpec((1, HID), lambda i: (0, 0)),
            pl.BlockSpec((HID, HID), lambda i: (0, 0)),
            pl.BlockSpec((1, HID), lambda i: (0, 0)),
            pl.BlockSpec((HID, 1), lambda i: (0, 0)),
            pl.BlockSpec((1, 1), lambda i: (0, 0)),
        ],
        out_specs=pl.BlockSpec((bblk, ACT_DIM), lambda i: (i, 0)),
        out_shape=jax.ShapeDtypeStruct((B, ACT_DIM), f32),
        compiler_params=pltpu.CompilerParams(
            dimension_semantics=("parallel",)),
    )(obs, raw_actions, W1c, b1c.reshape(1, HID), W2c, b2c.reshape(1, HID),
      W3c, b3c.reshape(1, 1))
    return best


# speedup vs baseline: 2.1796x; 1.0843x over previous
import functools

import jax
import jax.numpy as jnp
from jax.experimental import pallas as pl
from jax.experimental.pallas import tpu as pltpu

B, OBS_DIM, ACT_DIM, NUM_ACTIONS, HID, KNN = 1024, 128, 32, 100000, 256, 50
STD_NOISE, CLIP_NOISE, ACT_LO, ACT_HI = 0.1, 0.5, -1.0, 1.0

N_PAD = 102400
DIST_BLK = 2048
NBLK = N_PAD // DIST_BLK
GRP = 16
NGRP = N_PAD // GRP
NPAIR = NGRP // 2
NCAND = 2 * KNN * GRP
PAD_VAL = 1000.0


def _actor_body(obs, noise, w1, b1, w2, b2, w3, b3, proto_out):
    h = jax.nn.relu(jnp.dot(obs[...], w1[...]) + b1[...])
    h = jax.nn.relu(jnp.dot(h, w2[...]) + b2[...])
    p = jnp.tanh(jnp.dot(h, w3[...]) + b3[...])
    smooth = jnp.clip(noise[...] * STD_NOISE, -CLIP_NOISE, CLIP_NOISE)
    proto_out[...] = jnp.clip(p + smooth, ACT_LO, ACT_HI)


def _dist_body(proto, psq, table, asq, dcols_out, gmin_out):
    s = jnp.dot(proto[...], table[...].T)
    d = psq[...] - 2.0 * s + asq[...]
    d3 = d.reshape(B, GRP, 128)
    dcols_out[...] = d3
    gmin_out[...] = jnp.min(d3, axis=1)


def _select_body(g_ref, gid_out):
    blk, width = g_ref.shape
    g = g_ref[...]
    iota = jax.lax.broadcasted_iota(jnp.int32, (blk, width), 1)
    out_iota = jax.lax.broadcasted_iota(jnp.int32, (blk, 128), 1)

    def step(i, carry):
        g, gids = carry
        m = jnp.min(g, axis=1, keepdims=True)
        loc = jnp.min(jnp.where(g == m, iota, width), axis=1, keepdims=True)
        g = jnp.where(iota == loc, jnp.float32(jnp.inf), g)
        gids = jnp.where(out_iota == i, loc, gids)
        return g, gids

    _, gids = jax.lax.fori_loop(
        0, KNN, step, (g, jnp.zeros((blk, 128), jnp.int32)), unroll=4)
    gid_out[...] = gids


def _critic_body(obs, acts, w1, b1, w2, b2, w3, b3, best_out):
    blk = obs.shape[0]
    obs_t = jnp.broadcast_to(obs[...][:, None, :], (blk, KNN, OBS_DIM))
    ca = jnp.concatenate([obs_t, acts[...]], axis=-1)
    ca = ca.reshape(blk * KNN, OBS_DIM + ACT_DIM)
    q = jax.nn.relu(jnp.dot(ca, w1[...]) + b1[...])
    q = jax.nn.relu(jnp.dot(q, w2[...]) + b2[...])
    q = (jnp.dot(q, w3[...]) + b3[...]).reshape(blk, KNN)
    iota = jax.lax.broadcasted_iota(jnp.int32, (blk, KNN), 1)
    m = jnp.max(q, axis=1, keepdims=True)
    sel = jnp.where(q >= m, iota, KNN)
    bi = jnp.min(sel, axis=1, keepdims=True)
    onehot = (iota == bi).astype(jnp.float32)
    best_out[...] = jnp.sum(acts[...] * onehot[:, :, None], axis=1)


def kernel(obs, noise, actions_table, W1a, b1a, W2a, b2a, W3a, b3a,
           W1c, b1c, W2c, b2c, W3c, b3c):
    f32 = jnp.float32

    proto = pl.pallas_call(
        _actor_body,
        out_shape=jax.ShapeDtypeStruct((B, ACT_DIM), f32),
    )(obs, noise, W1a, b1a.reshape(1, HID), W2a, b2a.reshape(1, HID),
      W3a, b3a.reshape(1, ACT_DIM))

    table_pad = jnp.concatenate(
        [actions_table,
         jnp.full((N_PAD - NUM_ACTIONS, ACT_DIM), PAD_VAL, f32)], axis=0)
    psq = jnp.sum(proto * proto, axis=1, keepdims=True)
    asq = jnp.sum(table_pad * table_pad, axis=1)[None, :]

    dcols, gmin = pl.pallas_call(
        _dist_body,
        grid=(NBLK,),
        in_specs=[
            pl.BlockSpec((B, ACT_DIM), lambda i: (0, 0)),
            pl.BlockSpec((B, 1), lambda i: (0, 0)),
            pl.BlockSpec((DIST_BLK, ACT_DIM), lambda i: (i, 0)),
            pl.BlockSpec((1, DIST_BLK), lambda i: (0, i)),
        ],
        out_specs=[
            pl.BlockSpec((B, GRP, 128), lambda i: (0, 0, i)),
            pl.BlockSpec((B, 128), lambda i: (0, i)),
        ],
        out_shape=[
            jax.ShapeDtypeStruct((B, GRP, NGRP), f32),
            jax.ShapeDtypeStruct((B, NGRP), f32),
        ],
        compiler_params=pltpu.CompilerParams(
            dimension_semantics=("parallel",)),
    )(proto, psq, table_pad, asq)

    pmin = jnp.minimum(gmin[:, :NPAIR], gmin[:, NPAIR:])
    sblk = 512
    pid_full = pl.pallas_call(
        _select_body,
        grid=(B // sblk,),
        in_specs=[pl.BlockSpec((sblk, NPAIR), lambda i: (i, 0))],
        out_specs=pl.BlockSpec((sblk, 128), lambda i: (i, 0)),
        out_shape=jax.ShapeDtypeStruct((B, 128), jnp.int32),
        compiler_params=pltpu.CompilerParams(
            dimension_semantics=("parallel",)),
    )(pmin)
    pid = pid_full[:, :KNN]

    gidx2 = jnp.concatenate([pid, pid + NPAIR], axis=1)
    cands = jnp.take_along_axis(dcols, gidx2[:, None, :], axis=2)
    c_idx2 = gidx2 // 128
    l_idx2 = gidx2 % 128
    base2 = c_idx2 * DIST_BLK + l_idx2
    nn = base2[:, None, :] + 128 * jnp.arange(GRP, dtype=jnp.int32)[None, :, None]
    cv = cands.reshape(B, NCAND)
    ni = nn.reshape(B, NCAND)

    pos_full = pl.pallas_call(
        _select_body,
        grid=(B // sblk,),
        in_specs=[pl.BlockSpec((sblk, NCAND), lambda i: (i, 0))],
        out_specs=pl.BlockSpec((sblk, 128), lambda i: (i, 0)),
        out_shape=jax.ShapeDtypeStruct((B, 128), jnp.int32),
        compiler_params=pltpu.CompilerParams(
            dimension_semantics=("parallel",)),
    )(cv)
    idx = jnp.take_along_axis(ni, pos_full[:, :KNN], axis=1)
    raw_actions = jnp.take(actions_table, idx, axis=0)

    bblk = 128
    best = pl.pallas_call(
        _critic_body,
        grid=(B // bblk,),
        in_specs=[
            pl.BlockSpec((bblk, OBS_DIM), lambda i: (i, 0)),
            pl.BlockSpec((bblk, KNN, ACT_DIM), lambda i: (i, 0, 0)),
            pl.BlockSpec((OBS_DIM + ACT_DIM, HID), lambda i: (0, 0)),
            pl.BlockSpec((1, HID), lambda i: (0, 0)),
            pl.BlockSpec((HID, HID), lambda i: (0, 0)),
            pl.BlockSpec((1, HID), lambda i: (0, 0)),
            pl.BlockSpec((HID, 1), lambda i: (0, 0)),
            pl.BlockSpec((1, 1), lambda i: (0, 0)),
        ],
        out_specs=pl.BlockSpec((bblk, ACT_DIM), lambda i: (i, 0)),
        out_shape=jax.ShapeDtypeStruct((B, ACT_DIM), f32),
        compiler_params=pltpu.CompilerParams(
            dimension_semantics=("parallel",)),
    )(obs, raw_actions, W1c, b1c.reshape(1, HID), W2c, b2c.reshape(1, HID),
      W3c, b3c.reshape(1, 1))
    return best
